# baseline (speedup 1.0000x reference)
import functools

import jax
import jax.numpy as jnp
from jax.experimental import pallas as pl
from jax.experimental.pallas import tpu as pltpu

_C = 1000
_BETA = 0.9999
_BATCH = 16384
_R = 1024


def _wce_kernel(x_ref, t_ref, loss_ref, counts_acc, s_acc, *, n_steps):
    g = pl.program_id(0)

    @pl.when(g == 0)
    def _init():
        counts_acc[...] = jnp.zeros_like(counts_acc)
        s_acc[...] = jnp.zeros_like(s_acc)

    x = x_ref[...]
    t = t_ref[0, 0, :]

    m = jnp.max(x, axis=1, keepdims=True)
    e = jnp.exp(x - m)

    lane = jax.lax.broadcasted_iota(jnp.int32, x.shape, 1)
    mask = (lane == t[:, None]).astype(jnp.float32)
    xm = x * mask

    ones_c = jnp.ones((_C, 1), dtype=jnp.float32)
    s = (e @ ones_c)[:, 0]
    tgt = (xm @ ones_c)[:, 0]
    lse = m[:, 0] + jnp.log(s)
    nll = lse - tgt

    lhs = jnp.stack([jnp.ones_like(nll), nll], axis=0)
    acc = lhs @ mask
    counts_acc[0, :] += acc[0, :]
    s_acc[0, :] += acc[1, :]

    @pl.when(g == n_steps - 1)
    def _finish():
        counts = counts_acc[0, :]
        safe = jnp.maximum(counts, 1.0)
        w = (1.0 - _BETA) / (1.0 - jnp.exp(safe * jnp.log(_BETA)))
        num = jnp.sum(w * s_acc[0, :])
        den = jnp.sum(w * counts)
        loss_ref[...] = (num / den).reshape(1, 1)


def kernel(outputs, targets):
    n_steps = _BATCH // _R
    t3 = targets.reshape(n_steps, 1, _R)
    out = pl.pallas_call(
        functools.partial(_wce_kernel, n_steps=n_steps),
        grid=(n_steps,),
        in_specs=[
            pl.BlockSpec((_R, _C), lambda g: (g, 0)),
            pl.BlockSpec((1, 1, _R), lambda g: (g, 0, 0)),
        ],
        out_specs=pl.BlockSpec((1, 1), lambda g: (0, 0)),
        out_shape=jax.ShapeDtypeStruct((1, 1), jnp.float32),
        scratch_shapes=[
            pltpu.VMEM((1, _C), jnp.float32),
            pltpu.VMEM((1, _C), jnp.float32),
        ],
    )(outputs, t3)
    return out[0, 0]


# speedup vs baseline: 1.0191x; 1.0191x over previous
import functools
import math

import jax
import jax.numpy as jnp
from jax import lax
from jax.experimental import pallas as pl
from jax.experimental.pallas import tpu as pltpu
from jax.experimental.pallas import tpu_sc as plsc

_C = 1000
_CP = 1024
_BETA = 0.9999
_BATCH = 16384
_R = 1024


def _nll_kernel(x_ref, t_ref, nll_ref):
    x = x_ref[...]
    t = t_ref[0, 0, :]

    m = jnp.max(x, axis=1, keepdims=True)
    s = jnp.sum(jnp.exp(x - m), axis=1, keepdims=True)
    lse = m[:, 0] + jnp.log(s[:, 0])

    lane = jax.lax.broadcasted_iota(jnp.int32, x.shape, 1)
    tgt = jnp.sum(jnp.where(lane == t[:, None], x, 0.0), axis=1)
    nll_ref[...] = (lse - tgt).reshape(_R // 128, 128)


def _sc_finish(t_hbm, nll_hbm, out_hbm, t_v, nll_v, ones_v, z_v,
               counts_sh, s_sh, counts_l, s_l, out_v):
    sid = lax.axis_index("s")
    rows_per_tile = 8

    @pl.when(sid == 0)
    def _zero_shared():
        for k in range(_CP // 16):
            z_v[pl.ds(k * 16, 16)] = jnp.zeros((16,), jnp.float32)
        pltpu.sync_copy(z_v, counts_sh)
        pltpu.sync_copy(z_v, s_sh)

    for k in range(8):
        ones_v[pl.ds(k * 16, 16)] = jnp.ones((16,), jnp.float32)
    base = sid * rows_per_tile
    pltpu.sync_copy(t_hbm.at[pl.ds(base, rows_per_tile)], t_v)
    pltpu.sync_copy(nll_hbm.at[pl.ds(base, rows_per_tile)], nll_v)

    plsc.subcore_barrier()

    for j in range(rows_per_tile):
        idx = t_v.at[j]
        pltpu.sync_copy(ones_v, counts_sh.at[idx], add=True)
        pltpu.sync_copy(nll_v.at[j], s_sh.at[idx], add=True)

    plsc.subcore_barrier()

    @pl.when(sid == 0)
    def _finish():
        pltpu.sync_copy(counts_sh, counts_l)
        pltpu.sync_copy(s_sh, s_l)
        log_beta = jnp.float32(math.log(_BETA))
        one = jnp.float32(1.0)

        def body(k, carry):
            num16, den16 = carry
            off = pl.multiple_of(k * 16, 16)
            c16 = counts_l[pl.ds(off, 16)]
            s16 = s_l[pl.ds(off, 16)]
            safe = jnp.maximum(c16, 1.0)
            w = (one - _BETA) / (one - jnp.exp(safe * log_beta))
            return num16 + w * s16, den16 + w * c16

        z16 = jnp.zeros((16,), jnp.float32)
        num16, den16 = lax.fori_loop(0, _CP // 16, body, (z16, z16))

        lane = lax.iota(jnp.int32, 16)
        for sh in (8, 4, 2, 1):
            rot = (lane + sh) & 15
            num16 = num16 + num16.at[rot].get(mode="promise_in_bounds")
            den16 = den16 + den16.at[rot].get(mode="promise_in_bounds")
        out_v[...] = num16 / den16
        pltpu.sync_copy(out_v, out_hbm)


def kernel(outputs, targets):
    n_steps = _BATCH // _R
    t3 = targets.reshape(n_steps, 1, _R)
    nll = pl.pallas_call(
        _nll_kernel,
        grid=(n_steps,),
        in_specs=[
            pl.BlockSpec((_R, _C), lambda g: (g, 0)),
            pl.BlockSpec((1, 1, _R), lambda g: (g, 0, 0)),
        ],
        out_specs=pl.BlockSpec((_R // 128, 128), lambda g: (g, 0)),
        out_shape=jax.ShapeDtypeStruct((_BATCH // 128, 128), jnp.float32),
    )(outputs, t3)

    t2 = targets.reshape(_BATCH // 128, 128)
    mesh = plsc.VectorSubcoreMesh(
        core_axis_name="c", subcore_axis_name="s", num_cores=1)
    finish = pl.kernel(
        _sc_finish,
        out_type=jax.ShapeDtypeStruct((16,), jnp.float32),
        mesh=mesh,
        scratch_types=[
            pltpu.VMEM((8, 128), jnp.int32),
            pltpu.VMEM((8, 128), jnp.float32),
            pltpu.VMEM((128,), jnp.float32),
            pltpu.VMEM((_CP,), jnp.float32),
            pltpu.VMEM_SHARED((_CP,), jnp.float32),
            pltpu.VMEM_SHARED((_CP,), jnp.float32),
            pltpu.VMEM((_CP,), jnp.float32),
            pltpu.VMEM((_CP,), jnp.float32),
            pltpu.VMEM((16,), jnp.float32),
        ],
    )
    out = finish(t2, nll)
    return out[0]
